# Initial kernel scaffold; baseline (speedup 1.0000x reference)
import jax
import jax.numpy as jnp
from jax.experimental import pallas as pl


def kernel(pos, edge_cell_shift, params, atom_types, edge_index):
    raise NotImplementedError("write your pallas kernel here")


# speedup vs baseline: 3.9804x; 3.9804x over previous
import jax
import jax.numpy as jnp
import numpy as np
from jax.experimental import pallas as pl
from jax.experimental.pallas import tpu as pltpu

C = 16
LMAX = 2
R_MAX = 5.0
P = 6.0
NB = 8
NLAYERS = 2

_SLICES = [(0, 1), (1, 4), (4, 9)]
_LMAP = [0, 1, 1, 1, 2, 2, 2, 2, 2]
_M = 9
_D = C * _M
_EB = 512


def _fwd_kernel(dst_ref, w0_ref, w1_ref, y_ref, xs_ref, agg_ref, scratch):
    @pl.when(pl.program_id(0) == 0)
    def _():
        agg_ref[...] = jnp.zeros_like(agg_ref)

    xs = xs_ref[...]
    s = xs[:, 0:C]
    sy = jnp.concatenate([s * y_ref[:, m:m + 1] for m in range(_M)], axis=1)
    scratch[...] = w0_ref[...] * sy + w1_ref[...] * xs

    def body(e, carry):
        d = dst_ref[e]
        row = scratch[pl.ds(e, 1), :]
        agg_ref[pl.ds(d, 1), :] = agg_ref[pl.ds(d, 1), :] + row
        return carry

    jax.lax.fori_loop(0, _EB, body, 0)


def _bwd_kernel(ge_ref, w0_ref, w1_ref, y_ref, xs_ref,
                dw0_ref, dw1_ref, dy_ref, dxs_ref):
    ge = ge_ref[...]
    w0 = w0_ref[...]
    xs = xs_ref[...]
    s = xs[:, 0:C]
    sy = jnp.concatenate([s * y_ref[:, m:m + 1] for m in range(_M)], axis=1)
    dw0_ref[...] = ge * sy
    dw1_ref[...] = ge * xs
    t = ge * w0
    dy_cols = []
    ds = jnp.zeros_like(s)
    for m in range(_M):
        tm = t[:, m * C:(m + 1) * C]
        dy_cols.append(jnp.sum(tm * s, axis=1, keepdims=True))
        ds = ds + tm * y_ref[:, m:m + 1]
    dy_ref[...] = jnp.concatenate(dy_cols, axis=1)
    dxs = ge * w1_ref[...]
    dxs_ref[...] = jnp.concatenate([dxs[:, 0:C] + ds, dxs[:, C:]], axis=1)


def _edge_spec():
    return pl.BlockSpec((_EB, _D), lambda i: (i, 0))


def _msg_scatter(w0e, w1e, Y, xs, dst, n_nodes):
    E = w0e.shape[0]
    return pl.pallas_call(
        _fwd_kernel,
        grid=(E // _EB,),
        in_specs=[
            pl.BlockSpec((_EB,), lambda i: (i,), memory_space=pltpu.SMEM),
            _edge_spec(),
            _edge_spec(),
            pl.BlockSpec((_EB, _M), lambda i: (i, 0)),
            _edge_spec(),
        ],
        out_specs=pl.BlockSpec((n_nodes, _D), lambda i: (0, 0)),
        out_shape=jax.ShapeDtypeStruct((n_nodes, _D), jnp.float32),
        scratch_shapes=[pltpu.VMEM((_EB, _D), jnp.float32)],
    )(dst, w0e, w1e, Y, xs)


def _msg_bwd(ge, w0e, w1e, Y, xs):
    E = w0e.shape[0]
    return pl.pallas_call(
        _bwd_kernel,
        grid=(E // _EB,),
        in_specs=[_edge_spec(), _edge_spec(), _edge_spec(),
                  pl.BlockSpec((_EB, _M), lambda i: (i, 0)), _edge_spec()],
        out_specs=[_edge_spec(), _edge_spec(),
                   pl.BlockSpec((_EB, _M), lambda i: (i, 0)), _edge_spec()],
        out_shape=[
            jax.ShapeDtypeStruct((E, _D), jnp.float32),
            jax.ShapeDtypeStruct((E, _D), jnp.float32),
            jax.ShapeDtypeStruct((E, _M), jnp.float32),
            jax.ShapeDtypeStruct((E, _D), jnp.float32),
        ],
    )(ge, w0e, w1e, Y, xs)


@jax.custom_vjp
def _msg_agg(w0e, w1e, Y, xs, dst):
    return _msg_scatter(w0e, w1e, Y, xs, dst, 10000)


def _msg_agg_fwd(w0e, w1e, Y, xs, dst):
    return _msg_agg(w0e, w1e, Y, xs, dst), (w0e, w1e, Y, xs, dst)


def _msg_agg_bwd(res, g):
    w0e, w1e, Y, xs, dst = res
    ge = jnp.take(g, dst, axis=0)
    dw0e, dw1e, dY, dxs = _msg_bwd(ge, w0e, w1e, Y, xs)
    return dw0e, dw1e, dY, dxs, None


_msg_agg.defvjp(_msg_agg_fwd, _msg_agg_bwd)


def _bessel(r):
    n = jnp.arange(1, NB + 1, dtype=jnp.float32)
    x = r / R_MAX
    pre = np.sqrt(2.0 / R_MAX)
    b = pre * jnp.sin(n[None, :] * jnp.pi * x[:, None]) / jnp.maximum(r[:, None], 1e-6)
    fc = 1.0 - ((P + 1.0) * (P + 2.0) / 2.0) * x ** P + P * (P + 2.0) * x ** (P + 1.0) - (P * (P + 1.0) / 2.0) * x ** (P + 2.0)
    fc = jnp.where(x < 1.0, fc, 0.0)
    return b * fc[:, None]


def _sph(vec, r):
    u = vec / jnp.maximum(r, 1e-6)[:, None]
    ux, uy, uz = u[:, 0], u[:, 1], u[:, 2]
    s3 = np.sqrt(3.0)
    s15 = np.sqrt(15.0)
    s5 = np.sqrt(5.0)
    return jnp.stack([
        jnp.ones_like(ux),
        s3 * ux, s3 * uy, s3 * uz,
        s15 * ux * uy, s15 * uy * uz, (s5 / 2.0) * (3.0 * uz * uz - 1.0),
        s15 * ux * uz, (s15 / 2.0) * (ux * ux - uy * uy)
    ], axis=1)


def _energy(pos, edge_cell_shift, params, atom_types, edge_index):
    ei = jnp.concatenate([edge_index, edge_index[::-1]], axis=1)
    shift = jnp.concatenate([edge_cell_shift, -edge_cell_shift], axis=0)
    src = ei[0].astype(jnp.int32)
    dst = ei[1].astype(jnp.int32)
    vec = pos[dst] - pos[src] + shift
    r = jnp.sqrt(jnp.sum(vec * vec, axis=-1) + 1e-12)
    emb = _bessel(r)
    Y = _sph(vec, r)
    scal = params['W_embed'][atom_types]
    f = jnp.zeros((pos.shape[0], C, _M), dtype=pos.dtype).at[:, :, 0].set(scal)
    for l in range(NLAYERS):
        h = jax.nn.silu(emb @ params['W0_%d' % l] + params['b0_%d' % l])
        h = jax.nn.silu(h @ params['W1_%d' % l] + params['b1_%d' % l])
        w = (h @ params['W2_%d' % l]).reshape(-1, C, LMAX + 1, 2)
        w0e = w[:, :, _LMAP, 0].transpose(0, 2, 1).reshape(-1, _D)
        w1e = w[:, :, _LMAP, 1].transpose(0, 2, 1).reshape(-1, _D)
        xs = f[src].transpose(0, 2, 1).reshape(-1, _D)
        agg144 = _msg_agg(w0e, w1e, Y, xs, dst)
        agg = agg144.reshape(-1, _M, C).transpose(0, 2, 1)
        mixed = [jnp.einsum('ncm,cd->ndm', agg[:, :, a:b], params['Wself_%d' % l][li])
                 for li, (a, b) in enumerate(_SLICES)]
        f = f + jnp.concatenate(mixed, axis=2)
        s = f[:, :, 0]
        g = jax.nn.sigmoid(s @ params['Wgate_%d' % l]).reshape(-1, C, 2)
        f = jnp.concatenate([jax.nn.silu(s)[:, :, None],
                             f[:, :, 1:4] * g[:, :, 0:1],
                             f[:, :, 4:9] * g[:, :, 1:2]], axis=2)
    hnode = f[:, :, 0] @ params['W_hid']
    return jnp.sum(hnode @ params['W_out'])


def kernel(pos, edge_cell_shift, params, atom_types, edge_index):
    e, grads = jax.value_and_grad(_energy, argnums=0)(
        pos, edge_cell_shift, params, atom_types, edge_index)
    return e, -grads


# speedup vs baseline: 4.0814x; 1.0254x over previous
import jax
import jax.numpy as jnp
import numpy as np
from jax.experimental import pallas as pl
from jax.experimental.pallas import tpu as pltpu

C = 16
LMAX = 2
R_MAX = 5.0
P = 6.0
NB = 8
NLAYERS = 2

_SLICES = [(0, 1), (1, 4), (4, 9)]
_LMAP = [0, 1, 1, 1, 2, 2, 2, 2, 2]
_M = 9
_D = C * _M
_EB = 512


def _fwd_kernel(dst_ref, w0_ref, w1_ref, y_ref, xs_ref, agg_ref, scratch):
    @pl.when(pl.program_id(0) == 0)
    def _():
        agg_ref[...] = jnp.zeros_like(agg_ref)

    xs = xs_ref[...]
    s = xs[:, 0:C]
    sy = jnp.concatenate([s * y_ref[:, m:m + 1] for m in range(_M)], axis=1)
    scratch[...] = w0_ref[...] * sy + w1_ref[...] * xs

    def body(e, carry):
        d = dst_ref[e]
        row = scratch[pl.ds(e, 1), :]
        agg_ref[pl.ds(d, 1), :] = agg_ref[pl.ds(d, 1), :] + row
        return carry

    jax.lax.fori_loop(0, _EB, body, 0, unroll=8)


def _bwd_kernel(ge_ref, w0_ref, w1_ref, y_ref, xs_ref,
                dw0_ref, dw1_ref, dy_ref, dxs_ref):
    ge = ge_ref[...]
    w0 = w0_ref[...]
    xs = xs_ref[...]
    s = xs[:, 0:C]
    sy = jnp.concatenate([s * y_ref[:, m:m + 1] for m in range(_M)], axis=1)
    dw0_ref[...] = ge * sy
    dw1_ref[...] = ge * xs
    t = ge * w0
    dy_cols = []
    ds = jnp.zeros_like(s)
    for m in range(_M):
        tm = t[:, m * C:(m + 1) * C]
        dy_cols.append(jnp.sum(tm * s, axis=1, keepdims=True))
        ds = ds + tm * y_ref[:, m:m + 1]
    dy_ref[...] = jnp.concatenate(dy_cols, axis=1)
    dxs = ge * w1_ref[...]
    dxs_ref[...] = jnp.concatenate([dxs[:, 0:C] + ds, dxs[:, C:]], axis=1)


def _edge_spec():
    return pl.BlockSpec((_EB, _D), lambda i: (i, 0))


def _msg_scatter(w0e, w1e, Y, xs, dst, n_nodes):
    E = w0e.shape[0]
    return pl.pallas_call(
        _fwd_kernel,
        grid=(E // _EB,),
        in_specs=[
            pl.BlockSpec((_EB,), lambda i: (i,), memory_space=pltpu.SMEM),
            _edge_spec(),
            _edge_spec(),
            pl.BlockSpec((_EB, _M), lambda i: (i, 0)),
            _edge_spec(),
        ],
        out_specs=pl.BlockSpec((n_nodes, _D), lambda i: (0, 0)),
        out_shape=jax.ShapeDtypeStruct((n_nodes, _D), jnp.float32),
        scratch_shapes=[pltpu.VMEM((_EB, _D), jnp.float32)],
    )(dst, w0e, w1e, Y, xs)


def _msg_bwd(ge, w0e, w1e, Y, xs):
    E = w0e.shape[0]
    return pl.pallas_call(
        _bwd_kernel,
        grid=(E // _EB,),
        in_specs=[_edge_spec(), _edge_spec(), _edge_spec(),
                  pl.BlockSpec((_EB, _M), lambda i: (i, 0)), _edge_spec()],
        out_specs=[_edge_spec(), _edge_spec(),
                   pl.BlockSpec((_EB, _M), lambda i: (i, 0)), _edge_spec()],
        out_shape=[
            jax.ShapeDtypeStruct((E, _D), jnp.float32),
            jax.ShapeDtypeStruct((E, _D), jnp.float32),
            jax.ShapeDtypeStruct((E, _M), jnp.float32),
            jax.ShapeDtypeStruct((E, _D), jnp.float32),
        ],
    )(ge, w0e, w1e, Y, xs)


@jax.custom_vjp
def _msg_agg(w0e, w1e, Y, xs, dst):
    return _msg_scatter(w0e, w1e, Y, xs, dst, 10000)


def _msg_agg_fwd(w0e, w1e, Y, xs, dst):
    return _msg_agg(w0e, w1e, Y, xs, dst), (w0e, w1e, Y, xs, dst)


def _msg_agg_bwd(res, g):
    w0e, w1e, Y, xs, dst = res
    ge = jnp.take(g, dst, axis=0)
    dw0e, dw1e, dY, dxs = _msg_bwd(ge, w0e, w1e, Y, xs)
    return dw0e, dw1e, dY, dxs, None


_msg_agg.defvjp(_msg_agg_fwd, _msg_agg_bwd)


def _bessel(r):
    n = jnp.arange(1, NB + 1, dtype=jnp.float32)
    x = r / R_MAX
    pre = np.sqrt(2.0 / R_MAX)
    b = pre * jnp.sin(n[None, :] * jnp.pi * x[:, None]) / jnp.maximum(r[:, None], 1e-6)
    fc = 1.0 - ((P + 1.0) * (P + 2.0) / 2.0) * x ** P + P * (P + 2.0) * x ** (P + 1.0) - (P * (P + 1.0) / 2.0) * x ** (P + 2.0)
    fc = jnp.where(x < 1.0, fc, 0.0)
    return b * fc[:, None]


def _sph(vec, r):
    u = vec / jnp.maximum(r, 1e-6)[:, None]
    ux, uy, uz = u[:, 0], u[:, 1], u[:, 2]
    s3 = np.sqrt(3.0)
    s15 = np.sqrt(15.0)
    s5 = np.sqrt(5.0)
    return jnp.stack([
        jnp.ones_like(ux),
        s3 * ux, s3 * uy, s3 * uz,
        s15 * ux * uy, s15 * uy * uz, (s5 / 2.0) * (3.0 * uz * uz - 1.0),
        s15 * ux * uz, (s15 / 2.0) * (ux * ux - uy * uy)
    ], axis=1)


def _energy(pos, edge_cell_shift, params, atom_types, edge_index):
    ei = jnp.concatenate([edge_index, edge_index[::-1]], axis=1)
    shift = jnp.concatenate([edge_cell_shift, -edge_cell_shift], axis=0)
    src = ei[0].astype(jnp.int32)
    dst = ei[1].astype(jnp.int32)
    vec = pos[dst] - pos[src] + shift
    r = jnp.sqrt(jnp.sum(vec * vec, axis=-1) + 1e-12)
    emb = _bessel(r)
    Y = _sph(vec, r)
    scal = params['W_embed'][atom_types]
    f = jnp.zeros((pos.shape[0], C, _M), dtype=pos.dtype).at[:, :, 0].set(scal)
    for l in range(NLAYERS):
        h = jax.nn.silu(emb @ params['W0_%d' % l] + params['b0_%d' % l])
        h = jax.nn.silu(h @ params['W1_%d' % l] + params['b1_%d' % l])
        w = (h @ params['W2_%d' % l]).reshape(-1, C, LMAX + 1, 2)
        w0e = w[:, :, _LMAP, 0].transpose(0, 2, 1).reshape(-1, _D)
        w1e = w[:, :, _LMAP, 1].transpose(0, 2, 1).reshape(-1, _D)
        xs = f[src].transpose(0, 2, 1).reshape(-1, _D)
        agg144 = _msg_agg(w0e, w1e, Y, xs, dst)
        agg = agg144.reshape(-1, _M, C).transpose(0, 2, 1)
        mixed = [jnp.einsum('ncm,cd->ndm', agg[:, :, a:b], params['Wself_%d' % l][li])
                 for li, (a, b) in enumerate(_SLICES)]
        f = f + jnp.concatenate(mixed, axis=2)
        s = f[:, :, 0]
        g = jax.nn.sigmoid(s @ params['Wgate_%d' % l]).reshape(-1, C, 2)
        f = jnp.concatenate([jax.nn.silu(s)[:, :, None],
                             f[:, :, 1:4] * g[:, :, 0:1],
                             f[:, :, 4:9] * g[:, :, 1:2]], axis=2)
    hnode = f[:, :, 0] @ params['W_hid']
    return jnp.sum(hnode @ params['W_out'])


def kernel(pos, edge_cell_shift, params, atom_types, edge_index):
    e, grads = jax.value_and_grad(_energy, argnums=0)(
        pos, edge_cell_shift, params, atom_types, edge_index)
    return e, -grads


# speedup vs baseline: 7.5153x; 1.8413x over previous
import jax
import jax.numpy as jnp
import numpy as np
from jax.experimental import pallas as pl
from jax.experimental.pallas import tpu as pltpu

C = 16
LMAX = 2
R_MAX = 5.0
P = 6.0
NB = 8
NLAYERS = 2

_SLICES = [(0, 1), (1, 4), (4, 9)]
_LMAP = [0, 1, 1, 1, 2, 2, 2, 2, 2]
_M = 9
_D = C * _M
_EB = 512


def _fwd_kernel(dst_ref, w0_ref, w1_ref, y_ref, xs_ref, agg_ref, scratch):
    @pl.when(pl.program_id(0) == 0)
    def _():
        agg_ref[...] = jnp.zeros_like(agg_ref)

    xs = xs_ref[...]
    s = xs[:, 0:C]
    sy = jnp.concatenate([s * y_ref[:, m:m + 1] for m in range(_M)], axis=1)
    scratch[...] = w0_ref[...] * sy + w1_ref[...] * xs

    def body(e, carry):
        d = dst_ref[e]
        row = scratch[pl.ds(e, 1), :]
        agg_ref[pl.ds(d, 1), :] = agg_ref[pl.ds(d, 1), :] + row
        return carry

    jax.lax.fori_loop(0, _EB, body, 0, unroll=8)


def _bwd_kernel(ge_ref, w0_ref, w1_ref, y_ref, xs_ref,
                dw0_ref, dw1_ref, dy_ref, dxs_ref):
    ge = ge_ref[...]
    w0 = w0_ref[...]
    xs = xs_ref[...]
    s = xs[:, 0:C]
    sy = jnp.concatenate([s * y_ref[:, m:m + 1] for m in range(_M)], axis=1)
    dw0_ref[...] = ge * sy
    dw1_ref[...] = ge * xs
    t = ge * w0
    dy_cols = []
    ds = jnp.zeros_like(s)
    for m in range(_M):
        tm = t[:, m * C:(m + 1) * C]
        dy_cols.append(jnp.sum(tm * s, axis=1, keepdims=True))
        ds = ds + tm * y_ref[:, m:m + 1]
    dy_ref[...] = jnp.concatenate(dy_cols, axis=1)
    dxs = ge * w1_ref[...]
    dxs_ref[...] = jnp.concatenate([dxs[:, 0:C] + ds, dxs[:, C:]], axis=1)


def _edge_spec():
    return pl.BlockSpec((_EB, _D), lambda i: (i, 0))


def _msg_scatter(w0e, w1e, Y, xs, dst, n_nodes):
    E = w0e.shape[0]
    return pl.pallas_call(
        _fwd_kernel,
        grid=(E // _EB,),
        in_specs=[
            pl.BlockSpec((_EB,), lambda i: (i,), memory_space=pltpu.SMEM),
            _edge_spec(),
            _edge_spec(),
            pl.BlockSpec((_EB, _M), lambda i: (i, 0)),
            _edge_spec(),
        ],
        out_specs=pl.BlockSpec((n_nodes, _D), lambda i: (0, 0)),
        out_shape=jax.ShapeDtypeStruct((n_nodes, _D), jnp.float32),
        scratch_shapes=[pltpu.VMEM((_EB, _D), jnp.float32)],
    )(dst, w0e, w1e, Y, xs)


def _msg_bwd(ge, w0e, w1e, Y, xs):
    E = w0e.shape[0]
    return pl.pallas_call(
        _bwd_kernel,
        grid=(E // _EB,),
        in_specs=[_edge_spec(), _edge_spec(), _edge_spec(),
                  pl.BlockSpec((_EB, _M), lambda i: (i, 0)), _edge_spec()],
        out_specs=[_edge_spec(), _edge_spec(),
                   pl.BlockSpec((_EB, _M), lambda i: (i, 0)), _edge_spec()],
        out_shape=[
            jax.ShapeDtypeStruct((E, _D), jnp.float32),
            jax.ShapeDtypeStruct((E, _D), jnp.float32),
            jax.ShapeDtypeStruct((E, _M), jnp.float32),
            jax.ShapeDtypeStruct((E, _D), jnp.float32),
        ],
    )(ge, w0e, w1e, Y, xs)


@jax.custom_vjp
def _msg_agg(w0e, w1e, Y, xs, dst):
    return _msg_scatter(w0e, w1e, Y, xs, dst, 10000)


def _msg_agg_fwd(w0e, w1e, Y, xs, dst):
    return _msg_agg(w0e, w1e, Y, xs, dst), (w0e, w1e, Y, xs, dst)


def _msg_agg_bwd(res, g):
    w0e, w1e, Y, xs, dst = res
    ge = jnp.take(g, dst, axis=0)
    dw0e, dw1e, dY, dxs = _msg_bwd(ge, w0e, w1e, Y, xs)
    return dw0e, dw1e, dY, dxs, None


_msg_agg.defvjp(_msg_agg_fwd, _msg_agg_bwd)


def _scatter_kernel(idx_ref, rows_ref, out_ref):
    @pl.when(pl.program_id(0) == 0)
    def _():
        out_ref[...] = jnp.zeros_like(out_ref)

    def body(e, carry):
        d = idx_ref[e]
        out_ref[pl.ds(d, 1), :] = out_ref[pl.ds(d, 1), :] + rows_ref[pl.ds(e, 1), :]
        return carry

    jax.lax.fori_loop(0, _EB, body, 0, unroll=8)


def _row_scatter(rows, idx, n):
    E, D = rows.shape
    return pl.pallas_call(
        _scatter_kernel,
        grid=(E // _EB,),
        in_specs=[
            pl.BlockSpec((_EB,), lambda i: (i,), memory_space=pltpu.SMEM),
            pl.BlockSpec((_EB, D), lambda i: (i, 0)),
        ],
        out_specs=pl.BlockSpec((n, D), lambda i: (0, 0)),
        out_shape=jax.ShapeDtypeStruct((n, D), jnp.float32),
    )(idx, rows)


@jax.custom_vjp
def _gather_rows(x, idx):
    return jnp.take(x, idx, axis=0)


def _gather_rows_fwd(x, idx):
    return jnp.take(x, idx, axis=0), (idx, x.shape[0])


def _gather_rows_bwd(res, g):
    idx, n = res
    return _row_scatter(g, idx, n), None


_gather_rows.defvjp(_gather_rows_fwd, _gather_rows_bwd)


def _bessel(r):
    n = jnp.arange(1, NB + 1, dtype=jnp.float32)
    x = r / R_MAX
    pre = np.sqrt(2.0 / R_MAX)
    b = pre * jnp.sin(n[None, :] * jnp.pi * x[:, None]) / jnp.maximum(r[:, None], 1e-6)
    fc = 1.0 - ((P + 1.0) * (P + 2.0) / 2.0) * x ** P + P * (P + 2.0) * x ** (P + 1.0) - (P * (P + 1.0) / 2.0) * x ** (P + 2.0)
    fc = jnp.where(x < 1.0, fc, 0.0)
    return b * fc[:, None]


def _sph(vec, r):
    u = vec / jnp.maximum(r, 1e-6)[:, None]
    ux, uy, uz = u[:, 0], u[:, 1], u[:, 2]
    s3 = np.sqrt(3.0)
    s15 = np.sqrt(15.0)
    s5 = np.sqrt(5.0)
    return jnp.stack([
        jnp.ones_like(ux),
        s3 * ux, s3 * uy, s3 * uz,
        s15 * ux * uy, s15 * uy * uz, (s5 / 2.0) * (3.0 * uz * uz - 1.0),
        s15 * ux * uz, (s15 / 2.0) * (ux * ux - uy * uy)
    ], axis=1)


def _energy(pos, edge_cell_shift, params, atom_types, edge_index):
    ei = jnp.concatenate([edge_index, edge_index[::-1]], axis=1)
    shift = jnp.concatenate([edge_cell_shift, -edge_cell_shift], axis=0)
    src = ei[0].astype(jnp.int32)
    dst = ei[1].astype(jnp.int32)
    vec = _gather_rows(pos, dst) - _gather_rows(pos, src) + shift
    r = jnp.sqrt(jnp.sum(vec * vec, axis=-1) + 1e-12)
    emb = _bessel(r)
    Y = _sph(vec, r)
    scal = params['W_embed'][atom_types]
    f = jnp.zeros((pos.shape[0], C, _M), dtype=pos.dtype).at[:, :, 0].set(scal)
    for l in range(NLAYERS):
        h = jax.nn.silu(emb @ params['W0_%d' % l] + params['b0_%d' % l])
        h = jax.nn.silu(h @ params['W1_%d' % l] + params['b1_%d' % l])
        w = (h @ params['W2_%d' % l]).reshape(-1, C, LMAX + 1, 2)
        w0e = w[:, :, _LMAP, 0].transpose(0, 2, 1).reshape(-1, _D)
        w1e = w[:, :, _LMAP, 1].transpose(0, 2, 1).reshape(-1, _D)
        xs = _gather_rows(f.transpose(0, 2, 1).reshape(-1, _D), src)
        agg144 = _msg_agg(w0e, w1e, Y, xs, dst)
        agg = agg144.reshape(-1, _M, C).transpose(0, 2, 1)
        mixed = [jnp.einsum('ncm,cd->ndm', agg[:, :, a:b], params['Wself_%d' % l][li])
                 for li, (a, b) in enumerate(_SLICES)]
        f = f + jnp.concatenate(mixed, axis=2)
        s = f[:, :, 0]
        g = jax.nn.sigmoid(s @ params['Wgate_%d' % l]).reshape(-1, C, 2)
        f = jnp.concatenate([jax.nn.silu(s)[:, :, None],
                             f[:, :, 1:4] * g[:, :, 0:1],
                             f[:, :, 4:9] * g[:, :, 1:2]], axis=2)
    hnode = f[:, :, 0] @ params['W_hid']
    return jnp.sum(hnode @ params['W_out'])


def kernel(pos, edge_cell_shift, params, atom_types, edge_index):
    e, grads = jax.value_and_grad(_energy, argnums=0)(
        pos, edge_cell_shift, params, atom_types, edge_index)
    return e, -grads
